# Initial kernel scaffold; baseline (speedup 1.0000x reference)
import jax
import jax.numpy as jnp
from jax.experimental import pallas as pl


def kernel(vis_features_first, vis_features_second, inputs_first, inputs_second, embeddings, W_text):
    raise NotImplementedError("write your pallas kernel here")


# speedup vs baseline: 1.3969x; 1.3969x over previous
import jax
import jax.numpy as jnp
from jax.experimental import pallas as pl
from jax.experimental.pallas import tpu as pltpu
from jax.experimental.pallas import tpu_sc as plsc

VOCAB = 100000
DIM = 128
L = 200
TEXT_OUT = 512
N_PROMPTS = 2
K = L * DIM

GATHER_WINDOW = 128
N_IDX_PAD = 512
K_TILE = 3200


def _sc_gather(embeddings, idx2d):
    n_rows = idx2d.shape[1]
    mesh = plsc.VectorSubcoreMesh(core_axis_name="core",
                                  subcore_axis_name="subcore")

    @pl.kernel(out_type=jax.ShapeDtypeStruct((n_rows, DIM), embeddings.dtype),
               mesh=mesh)
    def gather_kernel(x_hbm, i_hbm, o_hbm):
        def body(i_vmem, o_vmem):
            pltpu.sync_copy(x_hbm.at[i_vmem.at[0]], o_vmem)

        pltpu.emit_pipeline(
            body,
            grid=(n_rows // GATHER_WINDOW,),
            in_specs=[pl.BlockSpec((1, GATHER_WINDOW),
                                   index_map=lambda i: (0, i))],
            out_specs=[pl.BlockSpec((GATHER_WINDOW, DIM),
                                    index_map=lambda i: (i, 0))],
            core_axis_name="subcore",
            dimension_semantics=(pltpu.PARALLEL,),
        )(i_hbm, o_hbm)

    return gather_kernel(embeddings, idx2d)


def _mm_body(p_ref, v_ref, w_ref, o_ref):
    k = pl.program_id(0)

    @pl.when(k == 0)
    def _():
        o_ref[...] = jnp.zeros_like(o_ref)

    p = p_ref[...] + v_ref[...]
    o_ref[...] += jnp.dot(p, w_ref[...], preferred_element_type=jnp.float32)


def _fused_matmul(p, v, w):
    return pl.pallas_call(
        _mm_body,
        grid=(K // K_TILE,),
        in_specs=[
            pl.BlockSpec((N_PROMPTS, K_TILE), lambda k: (0, k)),
            pl.BlockSpec((N_PROMPTS, K_TILE), lambda k: (0, k)),
            pl.BlockSpec((K_TILE, TEXT_OUT), lambda k: (k, 0)),
        ],
        out_specs=pl.BlockSpec((N_PROMPTS, TEXT_OUT), lambda k: (0, 0)),
        out_shape=jax.ShapeDtypeStruct((N_PROMPTS, TEXT_OUT), jnp.float32),
    )(p, v, w)


def kernel(vis_features_first, vis_features_second, inputs_first,
           inputs_second, embeddings, W_text):
    pad = jnp.zeros((N_IDX_PAD - N_PROMPTS * L,), jnp.int32)
    idx = jnp.concatenate([inputs_first.astype(jnp.int32),
                           inputs_second.astype(jnp.int32), pad])
    idx2d = idx.reshape(1, N_IDX_PAD)
    gathered = _sc_gather(embeddings, idx2d)
    p = gathered[:N_PROMPTS * L].reshape(N_PROMPTS, K)
    v = jnp.concatenate([vis_features_first, vis_features_second], axis=0)
    out = _fused_matmul(p, v, W_text)
    return (out[0:1], out[1:2])
